# baseline (speedup 1.0000x reference)
import functools

import jax
import jax.numpy as jnp
from jax import lax
from jax.experimental import pallas as pl
from jax.experimental.pallas import tpu as pltpu
from jax.experimental.pallas import tpu_sc as plsc

B, L, V = 4096, 20, 1000
C = V * L
NW = 32
RPW = B // NW
RPB = 1
NBUF = 4
RPS = RPB * NBUF

_mesh = plsc.VectorSubcoreMesh(core_axis_name="c", subcore_axis_name="s")


@functools.partial(
    pl.kernel,
    out_type=jax.ShapeDtypeStruct((B, C), jnp.float32),
    mesh=_mesh,
    compiler_params=pltpu.CompilerParams(needs_layout_passes=False),
    scratch_types=[
        pltpu.VMEM((RPW * L,), jnp.int32),
        pltpu.VMEM((RPB, C), jnp.float32),
        pltpu.VMEM((RPB, C), jnp.float32),
        pltpu.VMEM((RPB, C), jnp.float32),
        pltpu.VMEM((RPB, C), jnp.float32),
        pltpu.SemaphoreType.DMA,
        pltpu.SemaphoreType.DMA,
        pltpu.SemaphoreType.DMA,
        pltpu.SemaphoreType.DMA,
    ],
)
def _sc_one_hot(scaled_hbm, out_hbm, idx_v, buf0, buf1, buf2, buf3,
                sem0, sem1, sem2, sem3):
    wid = lax.axis_index("s") * 2 + lax.axis_index("c")
    base = wid * RPW
    pltpu.sync_copy(scaled_hbm.at[pl.ds(base * L, RPW * L)], idx_v)

    lane = lax.iota(jnp.int32, 16)
    zeros = jnp.zeros((16,), jnp.float32)
    ones = jnp.full((16,), 1.0, jnp.float32)
    hi_mask = lane >= 12
    bufs = (buf0, buf1, buf2, buf3)
    sems = (sem0, sem1, sem2, sem3)

    def memset(i, carry):
        for b in bufs:
            for jb in range(RPB):
                b[jb, pl.ds(i * 16, 16)] = zeros
        return carry

    lax.fori_loop(0, C // 16, memset, 0)

    def scatter_rows(buf, r, vals):
        for jb in range(RPB):
            off = (r + jb) * L + lane
            row = jnp.full((16,), jb, jnp.int32)
            g0 = plsc.load_gather(idx_v, [off])
            g1 = plsc.load_gather(idx_v, [off + 4])
            plsc.store_scatter(buf, [row, g0], vals)
            plsc.store_scatter(buf, [row, g1], vals, mask=hi_mask)

    def step(k, carry):
        for j in range(NBUF):
            r = k * RPS + j * RPB

            @pl.when(k > 0)
            def _(j=j, r=r):
                pltpu.make_async_copy(bufs[j], out_hbm.at[pl.ds(0, RPB)],
                                      sems[j]).wait()
                scatter_rows(bufs[j], r - RPS, zeros)
            scatter_rows(bufs[j], r, ones)
            pltpu.async_copy(bufs[j], out_hbm.at[pl.ds(base + r, RPB)], sems[j])
        return carry

    lax.fori_loop(0, RPW // RPS, step, 0)
    for j in range(NBUF):
        pltpu.make_async_copy(bufs[j], out_hbm.at[pl.ds(0, RPB)], sems[j]).wait()


def kernel(inpt, train_flag):
    scaled = inpt.astype(jnp.int32) * L + jnp.arange(L, dtype=jnp.int32)
    return _sc_one_hot(scaled.reshape(-1))
